# Initial kernel scaffold; baseline (speedup 1.0000x reference)
import jax
import jax.numpy as jnp
from jax.experimental import pallas as pl


def kernel(input, weight):
    raise NotImplementedError("write your pallas kernel here")


# speedup vs baseline: 1.1064x; 1.1064x over previous
import functools
import jax
import jax.numpy as jnp
from jax import lax
from jax.experimental import pallas as pl
from jax.experimental.pallas import tpu as pltpu
from jax.experimental.pallas import tpu_sc as plsc

NC = 2
NS = 16
NW = NC * NS

IDX_PER_GATHER = 128
GATHERS_PER_CHUNK = 8
CHUNK = IDX_PER_GATHER * GATHERS_PER_CHUNK


def _make_kernel(B, V, D):
    per_w = B // NW
    n_gather = per_w // IDX_PER_GATHER
    n_chunk = per_w // CHUNK

    mesh = plsc.VectorSubcoreMesh(core_axis_name="c", subcore_axis_name="s")

    @functools.partial(
        pl.kernel,
        out_type=jax.ShapeDtypeStruct((B, D), jnp.float32),
        mesh=mesh,
        scratch_types=[
            pltpu.VMEM((n_gather, IDX_PER_GATHER), jnp.int32),
            pltpu.VMEM((CHUNK, D), jnp.float32),
            pltpu.SemaphoreType.DMA,
        ],
        compiler_params=pltpu.CompilerParams(use_tc_tiling_on_sc=False),
    )
    def body(idx_hbm, table_hbm, out_hbm, idx_v, rows_v, sem):
        wid = lax.axis_index("s") * NC + lax.axis_index("c")
        base = wid * per_w
        pltpu.sync_copy(idx_hbm.at[wid], idx_v)

        def chunk_body(c, _):
            copies = []
            for g in range(GATHERS_PER_CHUNK):
                j = c * GATHERS_PER_CHUNK + g
                copies.append(
                    pltpu.async_copy(
                        table_hbm.at[idx_v.at[j]],
                        rows_v.at[pl.ds(g * IDX_PER_GATHER, IDX_PER_GATHER)],
                        sem,
                    )
                )
            for cp in copies:
                cp.wait()
            pltpu.sync_copy(rows_v, out_hbm.at[pl.ds(base + c * CHUNK, CHUNK)])
            return 0

        lax.fori_loop(0, n_chunk, chunk_body, 0)

    return body


@jax.jit
def kernel(input, weight):
    B, F = input.shape
    V, D = weight.shape
    total = B * F
    idx = input.reshape(NW, total // NW // IDX_PER_GATHER, IDX_PER_GATHER)
    idx = idx.astype(jnp.int32)
    out = _make_kernel(total, V, D)(idx, weight)
    return out.reshape(B, F, D)


# speedup vs baseline: 1.1114x; 1.0045x over previous
import functools
import jax
import jax.numpy as jnp
from jax import lax
from jax.experimental import pallas as pl
from jax.experimental.pallas import tpu as pltpu
from jax.experimental.pallas import tpu_sc as plsc

NC = 2
NS = 16
NW = NC * NS

IDX_PER_GATHER = 128
GATHERS_PER_CHUNK = 8
CHUNK = IDX_PER_GATHER * GATHERS_PER_CHUNK


def _make_kernel(B, V, D):
    per_w = B // NW
    n_gather = per_w // IDX_PER_GATHER
    n_chunk = per_w // CHUNK

    mesh = plsc.VectorSubcoreMesh(core_axis_name="c", subcore_axis_name="s")

    assert n_chunk % 2 == 0

    @functools.partial(
        pl.kernel,
        out_type=jax.ShapeDtypeStruct((B, D), jnp.float32),
        mesh=mesh,
        scratch_types=[
            pltpu.VMEM((n_gather, IDX_PER_GATHER), jnp.int32),
            pltpu.VMEM((CHUNK, D), jnp.float32),
            pltpu.VMEM((CHUNK, D), jnp.float32),
            pltpu.SemaphoreType.DMA,
            pltpu.SemaphoreType.DMA,
            pltpu.SemaphoreType.DMA,
            pltpu.SemaphoreType.DMA,
        ],
        compiler_params=pltpu.CompilerParams(use_tc_tiling_on_sc=False),
    )
    def body(idx_hbm, table_hbm, out_hbm, idx_v, rows0, rows1, sg0, sg1, so0, so1):
        wid = lax.axis_index("s") * NC + lax.axis_index("c")
        base = wid * per_w
        rows = (rows0, rows1)
        sg = (sg0, sg1)
        so = (so0, so1)
        pltpu.sync_copy(idx_hbm.at[wid], idx_v)

        def fire_gathers(c, slot):
            for g in range(GATHERS_PER_CHUNK):
                pltpu.async_copy(
                    table_hbm.at[idx_v.at[c * GATHERS_PER_CHUNK + g]],
                    rows[slot].at[pl.ds(g * IDX_PER_GATHER, IDX_PER_GATHER)],
                    sg[slot],
                )

        def wait_gathers(slot):
            pltpu.make_async_copy(
                out_hbm.at[pl.ds(0, CHUNK)], rows[slot], sg[slot]
            ).wait()

        def fire_out(c, slot):
            pltpu.async_copy(
                rows[slot], out_hbm.at[pl.ds(base + c * CHUNK, CHUNK)], so[slot]
            )

        def wait_out(c, slot):
            pltpu.make_async_copy(
                rows[slot], out_hbm.at[pl.ds(base + c * CHUNK, CHUNK)], so[slot]
            ).wait()

        def step(c, slot, other):
            wait_gathers(slot)
            fire_out(c, slot)
            pl.when(c >= 1)(lambda: wait_out(c - 1, other))
            pl.when(c + 1 < n_chunk)(lambda: fire_gathers(c + 1, other))

        fire_gathers(0, 0)

        def pair(i, _):
            c0 = i * 2
            step(c0, 0, 1)
            step(c0 + 1, 1, 0)
            return 0

        lax.fori_loop(0, n_chunk // 2, pair, 0)
        wait_out(n_chunk - 1, (n_chunk - 1) % 2)

    return body


@jax.jit
def kernel(input, weight):
    B, F = input.shape
    V, D = weight.shape
    total = B * F
    idx = input.reshape(NW, total // NW // IDX_PER_GATHER, IDX_PER_GATHER)
    idx = idx.astype(jnp.int32)
    out = _make_kernel(total, V, D)(idx, weight)
    return out.reshape(B, F, D)


# speedup vs baseline: 2.0367x; 1.8325x over previous
import functools
import jax
import jax.numpy as jnp
from jax import lax
from jax.experimental import pallas as pl
from jax.experimental.pallas import tpu as pltpu
from jax.experimental.pallas import tpu_sc as plsc

NC = 2
NS = 16
NW = NC * NS

ROUND = 256
PACK = 4
LANES = 128
D = 32
GROUPS = ROUND // 16


def _make_kernel(total, vocab):
    per_w = total // NW
    n_round = per_w // ROUND
    idx_rows_per_round = ROUND // LANES
    out_rows_per_round = ROUND // PACK
    idx_rows_per_w = per_w // LANES
    out_rows_per_w = per_w // PACK

    mesh = plsc.VectorSubcoreMesh(core_axis_name="c", subcore_axis_name="s")

    @functools.partial(
        pl.kernel,
        out_type=jax.ShapeDtypeStruct((total // PACK, LANES), jnp.float32),
        mesh=mesh,
        scratch_types=[
            pltpu.VMEM((idx_rows_per_round, LANES), jnp.int32),
            pltpu.VMEM((idx_rows_per_round, LANES), jnp.int32),
            pltpu.VMEM((idx_rows_per_round, LANES), jnp.int32),
            pltpu.VMEM((idx_rows_per_round, LANES), jnp.int32),
            pltpu.VMEM((idx_rows_per_round, LANES), jnp.int32),
            pltpu.VMEM((idx_rows_per_round, LANES), jnp.int32),
            pltpu.VMEM((ROUND, LANES), jnp.float32),
            pltpu.VMEM((ROUND, LANES), jnp.float32),
            pltpu.VMEM((out_rows_per_round, LANES), jnp.float32),
            pltpu.VMEM((out_rows_per_round, LANES), jnp.float32),
            pltpu.SemaphoreType.DMA,
            pltpu.SemaphoreType.DMA,
            pltpu.SemaphoreType.DMA,
            pltpu.SemaphoreType.DMA,
            pltpu.SemaphoreType.DMA,
            pltpu.SemaphoreType.DMA,
        ],
        compiler_params=pltpu.CompilerParams(
            use_tc_tiling_on_sc=False, needs_layout_passes=False),
    )
    def body(idx_hbm, table_hbm, out_hbm,
             idxb0, idxb1, pidx0, pidx1, ofb0, ofb1,
             pk0, pk1, ob0, ob1, si0, si1, sg0, sg1, so0, so1):
        wid = lax.axis_index("s") * NC + lax.axis_index("c")
        ibase = wid * idx_rows_per_w
        obase = wid * out_rows_per_w
        idxb = (idxb0, idxb1)
        pidx = (pidx0, pidx1)
        ofb = (ofb0, ofb1)
        pk = (pk0, pk1)
        ob = (ob0, ob1)
        si = (si0, si1)
        sg = (sg0, sg1)
        so = (so0, so1)

        def fire_idx(r, slot):
            pltpu.async_copy(
                idx_hbm.at[pl.ds(ibase + r * idx_rows_per_round,
                                 idx_rows_per_round)],
                idxb[slot], si[slot])

        def wait_idx(slot):
            pltpu.make_async_copy(
                idx_hbm.at[pl.ds(0, idx_rows_per_round)], idxb[slot],
                si[slot]).wait()

        def prep(slot):
            for q in range(idx_rows_per_round):
                for g in range(LANES // 16):
                    v = idxb[slot][q, pl.ds(g * 16, 16)]
                    pidx[slot][q, pl.ds(g * 16, 16)] = v >> 2
                    ofb[slot][q, pl.ds(g * 16, 16)] = (v & 3) * D

        def fire_gathers(slot):
            for s in range(idx_rows_per_round):
                pltpu.async_copy(
                    table_hbm.at[pidx[slot].at[s]],
                    pk[slot].at[pl.ds(s * LANES, LANES)], sg[slot])

        def wait_gathers(slot):
            pltpu.make_async_copy(
                table_hbm.at[pl.ds(0, ROUND)], pk[slot], sg[slot]).wait()

        def extract(slot):
            lane = lax.iota(jnp.int32, 16)
            for g in range(GROUPS):
                off = ofb[slot][g // 8, pl.ds((g % 8) * 16, 16)]
                lj = lane + g * 16
                orow = lj >> 2
                ocol0 = (lj & 3) * D
                for c in range(D):
                    vals = plsc.load_gather(pk[slot], [lj, off + c])
                    plsc.store_scatter(ob[slot], [orow, ocol0 + c], vals)

        def fire_out(r, slot):
            pltpu.async_copy(
                ob[slot],
                out_hbm.at[pl.ds(obase + r * out_rows_per_round,
                                 out_rows_per_round)], so[slot])

        def wait_out(slot):
            pltpu.make_async_copy(
                ob[slot], out_hbm.at[pl.ds(0, out_rows_per_round)],
                so[slot]).wait()

        def step(r, slot, other):
            def advance():
                wait_idx(other)
                prep(other)
            pl.when(r + 1 < n_round)(advance)
            wait_gathers(slot)
            pl.when(r + 1 < n_round)(lambda: fire_gathers(other))
            pl.when(r + 2 < n_round)(lambda: fire_idx(r + 2, slot))
            pl.when(r >= 2)(lambda: wait_out(slot))
            extract(slot)
            fire_out(r, slot)

        fire_idx(0, 0)
        wait_idx(0)
        prep(0)
        fire_gathers(0)
        fire_idx(1, 1)

        def pair(i, _):
            r0 = i * 2
            step(r0, 0, 1)
            step(r0 + 1, 1, 0)
            return 0

        lax.fori_loop(0, n_round // 2, pair, 0)
        wait_out(0)
        wait_out(1)

    return body


@jax.jit
def kernel(input, weight):
    B, F = input.shape
    V, _ = weight.shape
    total = B * F
    idx = input.astype(jnp.int32).reshape(total // LANES, LANES)
    wpk = weight.reshape(V // PACK, LANES)
    out = _make_kernel(total, V)(idx, wpk)
    return out.reshape(B, F, D)


# speedup vs baseline: 3.6869x; 1.8102x over previous
import functools
import jax
import jax.numpy as jnp
from jax import lax
from jax.experimental import pallas as pl
from jax.experimental.pallas import tpu as pltpu
from jax.experimental.pallas import tpu_sc as plsc

NC = 2
NS = 16
NW = NC * NS

ROUND = 256
PACK = 4
LANES = 128
D = 32
GROUPS = ROUND // 16


def _make_kernel(total, vocab):
    per_w = total // NW
    n_round = per_w // ROUND
    idx_rows_per_round = ROUND // LANES
    out_rows_per_round = ROUND // PACK
    idx_rows_per_w = per_w // LANES
    out_rows_per_w = per_w // PACK

    mesh = plsc.VectorSubcoreMesh(core_axis_name="c", subcore_axis_name="s")

    @functools.partial(
        pl.kernel,
        out_type=jax.ShapeDtypeStruct((total // PACK, LANES), jnp.float32),
        mesh=mesh,
        scratch_types=[
            pltpu.VMEM((idx_rows_per_round, LANES), jnp.int32),
            pltpu.VMEM((idx_rows_per_round, LANES), jnp.int32),
            pltpu.VMEM((idx_rows_per_round, LANES), jnp.int32),
            pltpu.VMEM((idx_rows_per_round, LANES), jnp.int32),
            pltpu.VMEM((idx_rows_per_round, LANES), jnp.int32),
            pltpu.VMEM((idx_rows_per_round, LANES), jnp.int32),
            pltpu.VMEM((ROUND, LANES), jnp.float32),
            pltpu.VMEM((ROUND, LANES), jnp.float32),
            pltpu.VMEM((out_rows_per_round, LANES), jnp.float32),
            pltpu.VMEM((out_rows_per_round, LANES), jnp.float32),
            pltpu.SemaphoreType.DMA,
            pltpu.SemaphoreType.DMA,
            pltpu.SemaphoreType.DMA,
            pltpu.SemaphoreType.DMA,
            pltpu.SemaphoreType.DMA,
            pltpu.SemaphoreType.DMA,
        ],
        compiler_params=pltpu.CompilerParams(
            use_tc_tiling_on_sc=False, needs_layout_passes=False),
    )
    def body(idx_hbm, table_hbm, out_hbm,
             idxb0, idxb1, pidx0, pidx1, ofb0, ofb1,
             pk0, pk1, ob0, ob1, si0, si1, sg0, sg1, so0, so1):
        wid = lax.axis_index("s") * NC + lax.axis_index("c")
        ibase = wid * idx_rows_per_w
        obase = wid * out_rows_per_w
        idxb = (idxb0, idxb1)
        pidx = (pidx0, pidx1)
        ofb = (ofb0, ofb1)
        pk = (pk0, pk1)
        ob = (ob0, ob1)
        si = (si0, si1)
        sg = (sg0, sg1)
        so = (so0, so1)

        def fire_idx(r, slot):
            pltpu.async_copy(
                idx_hbm.at[pl.ds(ibase + r * idx_rows_per_round,
                                 idx_rows_per_round)],
                idxb[slot], si[slot])

        def wait_idx(slot):
            pltpu.make_async_copy(
                idx_hbm.at[pl.ds(0, idx_rows_per_round)], idxb[slot],
                si[slot]).wait()

        def prep(slot):
            for q in range(idx_rows_per_round):
                for g in range(LANES // 16):
                    v = idxb[slot][q, pl.ds(g * 16, 16)]
                    pidx[slot][q, pl.ds(g * 16, 16)] = v >> 2
                    ofb[slot][q, pl.ds(g * 16, 16)] = (v & 3) * D

        def fire_gathers(slot):
            for s in range(idx_rows_per_round):
                pltpu.async_copy(
                    table_hbm.at[pidx[slot].at[s]],
                    pk[slot].at[pl.ds(s * LANES, LANES)], sg[slot])

        def wait_gathers(slot):
            pltpu.make_async_copy(
                table_hbm.at[pl.ds(0, ROUND)], pk[slot], sg[slot]).wait()

        def extract(slot):
            lane = lax.iota(jnp.int32, 16)

            def group(g, _):
                off = ofb[slot][g // 8, pl.ds((g % 8) * 16, 16)]
                lj = lane + g * 16
                orow = lj >> 2
                ocol0 = (lj & 3) * D
                for c in range(D):
                    rot = (lane + c) & (D - 1)
                    vals = plsc.load_gather(pk[slot], [lj, off + rot])
                    plsc.store_scatter(ob[slot], [orow, ocol0 + rot], vals)
                return 0

            lax.fori_loop(0, GROUPS, group, 0)

        def fire_out(r, slot):
            pltpu.async_copy(
                ob[slot],
                out_hbm.at[pl.ds(obase + r * out_rows_per_round,
                                 out_rows_per_round)], so[slot])

        def wait_out(slot):
            pltpu.make_async_copy(
                ob[slot], out_hbm.at[pl.ds(0, out_rows_per_round)],
                so[slot]).wait()

        def step(r, slot, other):
            def advance():
                wait_idx(other)
                prep(other)
            pl.when(r + 1 < n_round)(advance)
            wait_gathers(slot)
            pl.when(r + 1 < n_round)(lambda: fire_gathers(other))
            pl.when(r + 2 < n_round)(lambda: fire_idx(r + 2, slot))
            pl.when(r >= 2)(lambda: wait_out(slot))
            extract(slot)
            fire_out(r, slot)

        fire_idx(0, 0)
        wait_idx(0)
        prep(0)
        fire_gathers(0)
        fire_idx(1, 1)

        def pair(i, _):
            r0 = i * 2
            step(r0, 0, 1)
            step(r0 + 1, 1, 0)
            return 0

        lax.fori_loop(0, n_round // 2, pair, 0)
        wait_out(0)
        wait_out(1)

    return body


@jax.jit
def kernel(input, weight):
    B, F = input.shape
    V, _ = weight.shape
    total = B * F
    idx = input.astype(jnp.int32).reshape(total // LANES, LANES)
    wpk = weight.reshape(V // PACK, LANES)
    out = _make_kernel(total, V)(idx, wpk)
    return out.reshape(B, F, D)


# speedup vs baseline: 5.3062x; 1.4392x over previous
import functools
import jax
import jax.numpy as jnp
from jax import lax
from jax.experimental import pallas as pl
from jax.experimental.pallas import tpu as pltpu
from jax.experimental.pallas import tpu_sc as plsc

NC = 2
NS = 16
NW = NC * NS

ROUND = 256
PACK = 4
LANES = 128
D = 32
GROUPS = ROUND // 16
FIELDS = 100
FPAD = 104


def _make_kernel(total, vocab):
    per_w = total // NW
    n_round = per_w // ROUND
    idx_rows_per_round = ROUND // LANES
    idx_rows_per_w = per_w // LANES
    batch = total // FIELDS
    out_rows = batch * FPAD * PACK

    mesh = plsc.VectorSubcoreMesh(core_axis_name="c", subcore_axis_name="s")

    @functools.partial(
        pl.kernel,
        out_type=jax.ShapeDtypeStruct((out_rows, D), jnp.float32),
        mesh=mesh,
        scratch_types=[
            pltpu.VMEM((idx_rows_per_round, LANES), jnp.int32),
            pltpu.VMEM((idx_rows_per_round, LANES), jnp.int32),
            pltpu.VMEM((idx_rows_per_round, LANES), jnp.int32),
            pltpu.VMEM((idx_rows_per_round, LANES), jnp.int32),
            pltpu.VMEM((idx_rows_per_round, LANES), jnp.int32),
            pltpu.VMEM((idx_rows_per_round, LANES), jnp.int32),
            pltpu.VMEM((idx_rows_per_round, LANES), jnp.int32),
            pltpu.VMEM((idx_rows_per_round, LANES), jnp.int32),
            pltpu.VMEM((ROUND, LANES), jnp.float32),
            pltpu.VMEM((ROUND, LANES), jnp.float32),
            pltpu.VMEM((ROUND, D), jnp.float32),
            pltpu.VMEM((ROUND, D), jnp.float32),
            pltpu.SemaphoreType.DMA,
            pltpu.SemaphoreType.DMA,
            pltpu.SemaphoreType.DMA,
            pltpu.SemaphoreType.DMA,
            pltpu.SemaphoreType.DMA,
            pltpu.SemaphoreType.DMA,
        ],
        compiler_params=pltpu.CompilerParams(
            use_tc_tiling_on_sc=False, needs_layout_passes=False),
    )
    def body(idx_hbm, table_hbm, out_hbm,
             idxb0, idxb1, pidx0, pidx1, ofb0, ofb1, sidx0, sidx1,
             pk0, pk1, ob0, ob1, si0, si1, sg0, sg1, so0, so1):
        wid = lax.axis_index("s") * NC + lax.axis_index("c")
        ibase = wid * idx_rows_per_w
        osbase = wid * (per_w // FIELDS) * FPAD * PACK
        idxb = (idxb0, idxb1)
        pidx = (pidx0, pidx1)
        ofb = (ofb0, ofb1)
        sidx = (sidx0, sidx1)
        pk = (pk0, pk1)
        ob = (ob0, ob1)
        si = (si0, si1)
        sg = (sg0, sg1)
        so = (so0, so1)
        lane = lax.iota(jnp.int32, 16)

        def fire_idx(r, slot):
            pltpu.async_copy(
                idx_hbm.at[pl.ds(ibase + r * idx_rows_per_round,
                                 idx_rows_per_round)],
                idxb[slot], si[slot])

        def wait_idx(slot):
            pltpu.make_async_copy(
                idx_hbm.at[pl.ds(0, idx_rows_per_round)], idxb[slot],
                si[slot]).wait()

        def prep(r, slot):
            for q in range(idx_rows_per_round):
                for g in range(LANES // 16):
                    v = idxb[slot][q, pl.ds(g * 16, 16)]
                    pidx[slot][q, pl.ds(g * 16, 16)] = v >> 2
                    ofb[slot][q, pl.ds(g * 16, 16)] = (v & 3) * D
                    x = lane + (r * ROUND + q * LANES + g * 16)
                    bl = ((x >> 2) * 20972) >> 19
                    sidx[slot][q, pl.ds(g * 16, 16)] = (
                        osbase + x * PACK + bl * ((FPAD - FIELDS) * PACK))

        def fire_gathers(slot):
            for s in range(idx_rows_per_round):
                pltpu.async_copy(
                    table_hbm.at[pidx[slot].at[s]],
                    pk[slot].at[pl.ds(s * LANES, LANES)], sg[slot])

        def wait_gathers(slot):
            pltpu.make_async_copy(
                table_hbm.at[pl.ds(0, ROUND)], pk[slot], sg[slot]).wait()

        def extract(slot):
            def group(g, _):
                off = ofb[slot][g // 8, pl.ds((g % 8) * 16, 16)]
                lj = lane + g * 16
                for c in range(D):
                    rot = (lane + c) & (D - 1)
                    vals = plsc.load_gather(pk[slot], [lj, off + rot])
                    plsc.store_scatter(ob[slot], [lj, rot], vals)
                return 0

            lax.fori_loop(0, GROUPS, group, 0)

        def fire_out(slot):
            for s in range(idx_rows_per_round):
                pltpu.async_copy(
                    ob[slot].at[pl.ds(s * LANES, LANES)],
                    out_hbm.at[sidx[slot].at[s]], so[slot])

        def wait_out(slot):
            pltpu.make_async_copy(
                ob[slot], out_hbm.at[pl.ds(0, ROUND)], so[slot]).wait()

        def step(r, slot, other):
            def advance():
                wait_idx(other)
                pl.when(r >= 1)(lambda: wait_out(other))
                prep(r + 1, other)
            pl.when(r + 1 < n_round)(advance)
            wait_gathers(slot)
            pl.when(r + 1 < n_round)(lambda: fire_gathers(other))
            pl.when(r + 2 < n_round)(lambda: fire_idx(r + 2, slot))
            extract(slot)
            fire_out(slot)

        fire_idx(0, 0)
        wait_idx(0)
        prep(0, 0)
        fire_gathers(0)
        fire_idx(1, 1)

        def pair(i, _):
            r0 = i * 2
            step(r0, 0, 1)
            step(r0 + 1, 1, 0)
            return 0

        lax.fori_loop(0, n_round // 2, pair, 0)
        wait_out(0)
        wait_out(1)

    return body


@jax.jit
def kernel(input, weight):
    B, F = input.shape
    V, _ = weight.shape
    total = B * F
    idx = input.astype(jnp.int32).reshape(total // LANES, LANES)
    wpk = weight.reshape(V // PACK, LANES)
    out = _make_kernel(total, V)(idx, wpk)
    return out.reshape(B, FPAD, PACK * D)[:, :F, :D]
